# Initial kernel scaffold; baseline (speedup 1.0000x reference)
import jax
import jax.numpy as jnp
from jax.experimental import pallas as pl


def kernel(h, x, edge_index, params):
    raise NotImplementedError("write your pallas kernel here")


# speedup vs baseline: 2.6127x; 2.6127x over previous
import functools

import jax
import jax.numpy as jnp
from jax import lax
from jax.experimental import pallas as pl
from jax.experimental.pallas import tpu as pltpu
from jax.experimental.pallas import tpu_sc as plsc

N = 10000
E = 320000
H = 128

NC = 2
NS = 16
NW = NC * NS
EPW = E // NW
CHUNK = 80
NCHUNK = EPW // CHUNK
NG = N // 8
GPT = -(-NG // NS)

BE = 2560
NEB = E // BE
BN = 2000
NNB = N // BN

_mesh = plsc.VectorSubcoreMesh(core_axis_name="c", subcore_axis_name="s",
                               num_cores=NC, num_subcores=NS)


def _wid():
    return lax.axis_index("s") * NC + lax.axis_index("c")


def _make_gather2(W):
    @functools.partial(
        pl.kernel, mesh=_mesh,
        out_type=[jax.ShapeDtypeStruct((E, W), jnp.float32),
                  jax.ShapeDtypeStruct((E, W), jnp.float32)],
        scratch_types=[pltpu.VMEM((CHUNK,), jnp.int32),
                       pltpu.VMEM((CHUNK,), jnp.int32),
                       pltpu.VMEM((CHUNK, W), jnp.float32),
                       pltpu.VMEM((CHUNK, W), jnp.float32),
                       pltpu.SemaphoreType.DMA,
                       pltpu.SemaphoreType.DMA],
        name=f"sc_gather2_{W}",
    )
    def gather2(ta_hbm, tb_hbm, row_hbm, col_hbm, ga_hbm, gb_hbm,
                rowv, colv, bufa, bufb, sema, semb):
        base = _wid() * EPW

        def body(j, carry):
            e0 = base + j * CHUNK
            pltpu.sync_copy(row_hbm.at[pl.ds(e0, CHUNK)], rowv)
            pltpu.sync_copy(col_hbm.at[pl.ds(e0, CHUNK)], colv)
            ca = pltpu.async_copy(ta_hbm.at[rowv], bufa, sema)
            cb = pltpu.async_copy(tb_hbm.at[colv], bufb, semb)
            ca.wait()
            cb.wait()
            pltpu.sync_copy(bufa, ga_hbm.at[pl.ds(e0, CHUNK)])
            pltpu.sync_copy(bufb, gb_hbm.at[pl.ds(e0, CHUNK)])
            return carry

        lax.fori_loop(0, NCHUNK, body, 0)

    return gather2


_gather2_h = _make_gather2(H)


def _make_segsum(mode):
    out_type = [jax.ShapeDtypeStruct((NC, N, H), jnp.float32)]
    scratch = [pltpu.VMEM((NCHUNK, CHUNK), jnp.int32),
               pltpu.VMEM((CHUNK, H), jnp.float32),
               pltpu.VMEM((8, H), jnp.float32),
               pltpu.VMEM_SHARED((N, H), jnp.float32)]

    def body(*refs):
        if mode == "plain":
            (u_hbm, row2_hbm, zH_hbm, u2_hbm,
             idx2, ubuf, zbuf, accU) = refs
        else:
            (row2_hbm, zH_hbm, o_hbm, u2_hbm,
             idx2, ubuf, zbuf, accU) = refs

        cid = lax.axis_index("c")
        sid = lax.axis_index("s")
        wid = _wid()
        base = wid * EPW

        pltpu.sync_copy(row2_hbm.at[wid], idx2)
        pltpu.sync_copy(zH_hbm, zbuf)
        if mode == "ones":
            pltpu.sync_copy(o_hbm, ubuf)

        nk = (NG - sid + NS - 1) // NS

        def zcp(k, c):
            r0 = (k * NS + sid) * 8
            pltpu.sync_copy(zbuf, accU.at[pl.ds(r0, 8)])
            return c
        lax.fori_loop(0, nk, zcp, 0)

        plsc.subcore_barrier()

        def scat(j, c):
            if mode == "plain":
                e0 = base + j * CHUNK
                pltpu.sync_copy(u_hbm.at[pl.ds(e0, CHUNK)], ubuf)
            pltpu.sync_copy(ubuf, accU.at[idx2.at[j]], add=True)
            return c
        lax.fori_loop(0, NCHUNK, scat, 0)

        plsc.subcore_barrier()

        def wout(k, c):
            r0 = (k * NS + sid) * 8
            pltpu.sync_copy(accU.at[pl.ds(r0, 8)],
                            u2_hbm.at[cid, pl.ds(r0, 8)])
            return c
        lax.fori_loop(0, nk, wout, 0)

    return functools.partial(pl.kernel, mesh=_mesh, out_type=out_type,
                             scratch_types=scratch,
                             name=f"sc_segsum_{mode}")(body)


_segsum_plain = _make_segsum("plain")
_segsum_ones = _make_segsum("ones")


def _full(shape):
    return pl.BlockSpec(shape, lambda i: tuple(0 for _ in shape))


def _init_body(h_ref, wemb_ref, bemb_ref, w1a_ref, be1_ref, w1b_ref,
               h1_ref, a_ref, b_ref):
    h1 = jnp.dot(h_ref[...], wemb_ref[...],
                 preferred_element_type=jnp.float32) + bemb_ref[...]
    h1_ref[...] = h1
    a_ref[...] = jnp.dot(h1, w1a_ref[...],
                         preferred_element_type=jnp.float32) + be1_ref[...]
    b_ref[...] = jnp.dot(h1, w1b_ref[...], preferred_element_type=jnp.float32)


_init_call = pl.pallas_call(
    _init_body,
    grid=(NNB,),
    in_specs=[pl.BlockSpec((BN, H), lambda i: (i, 0)),
              _full((H, H)), _full((1, H)), _full((H, H)), _full((1, H)),
              _full((H, H))],
    out_specs=[pl.BlockSpec((BN, H), lambda i: (i, 0))] * 3,
    out_shape=[jax.ShapeDtypeStruct((N, H), jnp.float32)] * 3,
)


def _geom_body(xr_ref, xc_ref, g_ref):
    dx = xr_ref[:, 0:1] - xc_ref[:, 0:1]
    dy = xr_ref[:, 1:2] - xc_ref[:, 1:2]
    d = dx * dx + dy * dy
    rpx = dx * dx - dy * dy
    rpy = 2.0 * dx * dy
    z = jnp.zeros_like(xr_ref[:, 0:13])
    g_ref[...] = jnp.concatenate([d, rpx, rpy, z], axis=1)


_geom_call = pl.pallas_call(
    _geom_body,
    grid=(NEB,),
    in_specs=[pl.BlockSpec((BE, H), lambda i: (i, 0))] * 2,
    out_specs=pl.BlockSpec((BE, 16), lambda i: (i, 0)),
    out_shape=jax.ShapeDtypeStruct((E, 16), jnp.float32),
)


def _edge_plain_body(ga_ref, gb_ref, g_ref, wd_ref, u_ref):
    pre = ga_ref[...] + gb_ref[...] + g_ref[:, 0:1] * wd_ref[...]
    u_ref[...] = jnp.maximum(pre, 0.0)


_edge_plain = pl.pallas_call(
    _edge_plain_body,
    grid=(NEB,),
    in_specs=[pl.BlockSpec((BE, H), lambda i: (i, 0)),
              pl.BlockSpec((BE, H), lambda i: (i, 0)),
              pl.BlockSpec((BE, 16), lambda i: (i, 0)),
              _full((1, H))],
    out_specs=pl.BlockSpec((BE, H), lambda i: (i, 0)),
    out_shape=jax.ShapeDtypeStruct((E, H), jnp.float32),
)


def _edge_t_body(ga_ref, gb_ref, g_ref, wd_ref, wec_ref, bec_ref, wc2_ref,
                 bc2_ref, u_ref, r_ref):
    pre = ga_ref[...] + gb_ref[...] + g_ref[:, 0:1] * wd_ref[...]
    u = jnp.maximum(pre, 0.0)
    u_ref[...] = u
    z = jnp.maximum(jnp.dot(u, wec_ref[...],
                            preferred_element_type=jnp.float32) + bec_ref[...],
                    0.0)
    t = jnp.sum(z * wc2_ref[...], axis=1, keepdims=True) + bc2_ref[...]
    r_ref[...] = jnp.concatenate(
        [g_ref[:, 1:2] * t, g_ref[:, 2:3] * t,
         jnp.zeros((g_ref.shape[0], H - 2), jnp.float32)], axis=1)


_edge_t = pl.pallas_call(
    _edge_t_body,
    grid=(NEB,),
    in_specs=[pl.BlockSpec((BE, H), lambda i: (i, 0)),
              pl.BlockSpec((BE, H), lambda i: (i, 0)),
              pl.BlockSpec((BE, 16), lambda i: (i, 0)),
              _full((1, H)), _full((H, H)), _full((1, H)), _full((1, H)),
              _full((1, 1))],
    out_specs=[pl.BlockSpec((BE, H), lambda i: (i, 0)),
               pl.BlockSpec((BE, H), lambda i: (i, 0))],
    out_shape=[jax.ShapeDtypeStruct((E, H), jnp.float32),
               jax.ShapeDtypeStruct((E, H), jnp.float32)],
)


def _make_node(last):
    def body(h_ref, u2_ref, c2_ref, *rest):
        if last:
            s2_ref = rest[0]
            rest = rest[1:]
        (we2_ref, be2_ref, wn1a_ref, wn1b_ref,
         bn1_ref, wn2_ref, bn2_ref, wA_ref, bA_ref, wB_ref, *outs) = rest
        h = h_ref[...]
        U = u2_ref[0] + u2_ref[1]
        cnt = c2_ref[0, :, 0:1] + c2_ref[1, :, 0:1]
        m = jnp.dot(U, we2_ref[...],
                    preferred_element_type=jnp.float32) + cnt * be2_ref[...]
        q = jnp.maximum(
            jnp.dot(h, wn1a_ref[...], preferred_element_type=jnp.float32)
            + jnp.dot(m, wn1b_ref[...], preferred_element_type=jnp.float32)
            + bn1_ref[...], 0.0)
        hn = h + jnp.dot(q, wn2_ref[...],
                         preferred_element_type=jnp.float32) + bn2_ref[...]
        outs[0][...] = hn
        if not last:
            outs[1][...] = jnp.dot(hn, wA_ref[...],
                                   preferred_element_type=jnp.float32) + bA_ref[...]
            outs[2][...] = jnp.dot(hn, wB_ref[...],
                                   preferred_element_type=jnp.float32)
        else:
            qv = jnp.maximum(jnp.dot(hn, wA_ref[...],
                                     preferred_element_type=jnp.float32)
                             + bA_ref[...], 0.0)
            vv = jnp.dot(qv, wB_ref[...], preferred_element_type=jnp.float32)
            s = s2_ref[0, :, 0:2] + s2_ref[1, :, 0:2]
            v = vv + s / jnp.maximum(cnt, 1.0)
            nrm = jnp.sqrt(jnp.sum(v * v, axis=1, keepdims=True))
            outs[1][...] = v / jnp.maximum(nrm, 1e-12)

    big_spec = pl.BlockSpec((NC, BN, H), lambda i: (0, i, 0))
    if last:
        out_specs = [pl.BlockSpec((BN, H), lambda i: (i, 0)),
                     pl.BlockSpec((BN, 2), lambda i: (i, 0))]
        out_shape = [jax.ShapeDtypeStruct((N, H), jnp.float32),
                     jax.ShapeDtypeStruct((N, 2), jnp.float32)]
        wB_spec = _full((H, 2))
        extra = [big_spec]
    else:
        out_specs = [pl.BlockSpec((BN, H), lambda i: (i, 0))] * 3
        out_shape = [jax.ShapeDtypeStruct((N, H), jnp.float32)] * 3
        wB_spec = _full((H, H))
        extra = []

    return pl.pallas_call(
        body,
        grid=(NNB,),
        in_specs=[pl.BlockSpec((BN, H), lambda i: (i, 0)),
                  big_spec, big_spec] + extra +
                 [_full((H, H)), _full((1, H)), _full((H, H)), _full((H, H)),
                  _full((1, H)), _full((H, H)), _full((1, H)),
                  _full((H, H)), _full((1, H)), wB_spec],
        out_specs=out_specs,
        out_shape=out_shape,
    )


_node_mid = _make_node(False)
_node_last = _make_node(True)


def kernel(h, x, edge_index, params):
    p = params
    row = edge_index[0]
    col = edge_index[1]

    W1a = p['W_e1'][:H]
    W1b = p['W_e1'][H:2 * H]
    wd = p['W_e1'][2 * H].reshape(1, H)
    W_ec = p['W_e2'] @ p['W_c1']
    b_ec = (p['b_e2'] @ p['W_c1'] + p['b_c1']).reshape(1, H)
    wc2 = p['W_c2'].reshape(1, H)
    bc2 = p['b_c2'].reshape(1, 1)
    be1 = p['b_e1'].reshape(1, H)
    be2 = p['b_e2'].reshape(1, H)
    Wn1a = p['W_n1'][:H]
    Wn1b = p['W_n1'][H:]
    bn1 = p['b_n1'].reshape(1, H)
    bn2 = p['b_n2'].reshape(1, H)
    bv1 = p['b_v1'].reshape(1, H)

    xp = jnp.pad(x, ((0, 0), (0, H - 2)))
    row3 = row.reshape(NW, NCHUNK, CHUNK)
    zH = jnp.zeros((8, H), jnp.float32)
    oH = jnp.ones((CHUNK, H), jnp.float32)

    h1, A, B = _init_call(h, p['W_emb'], p['b_emb'].reshape(1, H),
                          W1a, be1, W1b)
    xr, xc = _gather2_h(xp, xp, row, col)
    geom = _geom_call(xr, xc)

    (C2,) = _segsum_ones(row3, zH, oH)

    v = None
    for l in range(4):
        ga, gb = _gather2_h(A, B, row, col)
        if l < 3:
            u = _edge_plain(ga, gb, geom, wd)
        else:
            u, r128 = _edge_t(ga, gb, geom, wd, W_ec, b_ec, wc2, bc2)
        (U2,) = _segsum_plain(u, row3, zH)
        if l < 3:
            h1, A, B = _node_mid(h1, U2, C2, p['W_e2'], be2, Wn1a, Wn1b,
                                 bn1, p['W_n2'], bn2, W1a, be1, W1b)
        else:
            (S2,) = _segsum_plain(r128, row3, zH)
            h1, v = _node_last(h1, U2, C2, S2, p['W_e2'], be2, Wn1a, Wn1b,
                               bn1, p['W_n2'], bn2, p['W_v1'], bv1, p['W_v2'])

    return (h1, x, v)


# speedup vs baseline: 2.9733x; 1.1380x over previous
import functools

import jax
import jax.numpy as jnp
from jax import lax
from jax.experimental import pallas as pl
from jax.experimental.pallas import tpu as pltpu
from jax.experimental.pallas import tpu_sc as plsc

N = 10000
E = 320000
H = 128

NC = 2
NS = 16
NW = NC * NS
EPW = E // NW
CHUNK = 80
NCHUNK = EPW // CHUNK
NG = N // 8
GPT = -(-NG // NS)

BE = 2560
NEB = E // BE
BN = 2000
NNB = N // BN

_mesh = plsc.VectorSubcoreMesh(core_axis_name="c", subcore_axis_name="s",
                               num_cores=NC, num_subcores=NS)


def _wid():
    return lax.axis_index("s") * NC + lax.axis_index("c")


def _make_gather2(W):
    @functools.partial(
        pl.kernel, mesh=_mesh,
        out_type=[jax.ShapeDtypeStruct((E, W), jnp.float32),
                  jax.ShapeDtypeStruct((E, W), jnp.float32)],
        scratch_types=[pltpu.VMEM((2, CHUNK), jnp.int32),
                       pltpu.VMEM((2, CHUNK), jnp.int32),
                       pltpu.VMEM((2, CHUNK, W), jnp.float32),
                       pltpu.VMEM((2, CHUNK, W), jnp.float32),
                       pltpu.SemaphoreType.DMA,
                       pltpu.SemaphoreType.DMA,
                       pltpu.SemaphoreType.DMA,
                       pltpu.SemaphoreType.DMA],
        name=f"sc_gather2_{W}",
    )
    def gather2(ta_hbm, tb_hbm, row_hbm, col_hbm, ga_hbm, gb_hbm,
                rowv, colv, bufa, bufb, sa0, sa1, sb0, sb1):
        base = _wid() * EPW
        sas = (sa0, sa1)
        sbs = (sb0, sb1)

        def start(j, b):
            e0 = base + j * CHUNK
            pltpu.sync_copy(row_hbm.at[pl.ds(e0, CHUNK)], rowv.at[b])
            pltpu.sync_copy(col_hbm.at[pl.ds(e0, CHUNK)], colv.at[b])
            pltpu.async_copy(ta_hbm.at[rowv.at[b]], bufa.at[b], sas[b])
            pltpu.async_copy(tb_hbm.at[colv.at[b]], bufb.at[b], sbs[b])

        def finish(j, b):
            e0 = base + j * CHUNK
            pltpu.make_async_copy(ta_hbm.at[rowv.at[b]], bufa.at[b],
                                  sas[b]).wait()
            pltpu.make_async_copy(tb_hbm.at[colv.at[b]], bufb.at[b],
                                  sbs[b]).wait()
            pltpu.sync_copy(bufa.at[b], ga_hbm.at[pl.ds(e0, CHUNK)])
            pltpu.sync_copy(bufb.at[b], gb_hbm.at[pl.ds(e0, CHUNK)])

        start(0, 0)

        def body(k, carry):
            j = 2 * k
            start(j + 1, 1)
            finish(j, 0)
            start(j + 2, 0)
            finish(j + 1, 1)
            return carry

        lax.fori_loop(0, (NCHUNK - 1) // 2, body, 0)
        finish(NCHUNK - 1, 0)

    return gather2


_gather2_h = _make_gather2(H)


def _make_segsum(mode):
    out_type = [jax.ShapeDtypeStruct((NC, N, H), jnp.float32)]
    scratch = [pltpu.VMEM((NCHUNK, CHUNK), jnp.int32),
               pltpu.VMEM((CHUNK, H), jnp.float32),
               pltpu.VMEM((8, H), jnp.float32),
               pltpu.VMEM_SHARED((N, H), jnp.float32)]

    def body(*refs):
        if mode == "plain":
            (u_hbm, row2_hbm, zH_hbm, u2_hbm,
             idx2, ubuf, zbuf, accU) = refs
        else:
            (row2_hbm, zH_hbm, o_hbm, u2_hbm,
             idx2, ubuf, zbuf, accU) = refs

        cid = lax.axis_index("c")
        sid = lax.axis_index("s")
        wid = _wid()
        base = wid * EPW

        pltpu.sync_copy(row2_hbm.at[wid], idx2)
        pltpu.sync_copy(zH_hbm, zbuf)
        if mode == "ones":
            pltpu.sync_copy(o_hbm, ubuf)

        nk = (NG - sid + NS - 1) // NS

        def zcp(k, c):
            r0 = (k * NS + sid) * 8
            pltpu.sync_copy(zbuf, accU.at[pl.ds(r0, 8)])
            return c
        lax.fori_loop(0, nk, zcp, 0)

        plsc.subcore_barrier()

        def scat(j, c):
            if mode == "plain":
                e0 = base + j * CHUNK
                pltpu.sync_copy(u_hbm.at[pl.ds(e0, CHUNK)], ubuf)
            pltpu.sync_copy(ubuf, accU.at[idx2.at[j]], add=True)
            return c
        lax.fori_loop(0, NCHUNK, scat, 0)

        plsc.subcore_barrier()

        def wout(k, c):
            r0 = (k * NS + sid) * 8
            pltpu.sync_copy(accU.at[pl.ds(r0, 8)],
                            u2_hbm.at[cid, pl.ds(r0, 8)])
            return c
        lax.fori_loop(0, nk, wout, 0)

    return functools.partial(pl.kernel, mesh=_mesh, out_type=out_type,
                             scratch_types=scratch,
                             name=f"sc_segsum_{mode}")(body)


_segsum_plain = _make_segsum("plain")
_segsum_ones = _make_segsum("ones")


def _full(shape):
    return pl.BlockSpec(shape, lambda i: tuple(0 for _ in shape))


def _init_body(h_ref, wemb_ref, bemb_ref, w1a_ref, be1_ref, w1b_ref,
               h1_ref, a_ref, b_ref):
    h1 = jnp.dot(h_ref[...], wemb_ref[...],
                 preferred_element_type=jnp.float32) + bemb_ref[...]
    h1_ref[...] = h1
    a_ref[...] = jnp.dot(h1, w1a_ref[...],
                         preferred_element_type=jnp.float32) + be1_ref[...]
    b_ref[...] = jnp.dot(h1, w1b_ref[...], preferred_element_type=jnp.float32)


_init_call = pl.pallas_call(
    _init_body,
    grid=(NNB,),
    in_specs=[pl.BlockSpec((BN, H), lambda i: (i, 0)),
              _full((H, H)), _full((1, H)), _full((H, H)), _full((1, H)),
              _full((H, H))],
    out_specs=[pl.BlockSpec((BN, H), lambda i: (i, 0))] * 3,
    out_shape=[jax.ShapeDtypeStruct((N, H), jnp.float32)] * 3,
)


def _geom_body(xr_ref, xc_ref, g_ref):
    dx = xr_ref[:, 0:1] - xc_ref[:, 0:1]
    dy = xr_ref[:, 1:2] - xc_ref[:, 1:2]
    d = dx * dx + dy * dy
    rpx = dx * dx - dy * dy
    rpy = 2.0 * dx * dy
    z = jnp.zeros_like(xr_ref[:, 0:13])
    g_ref[...] = jnp.concatenate([d, rpx, rpy, z], axis=1)


_geom_call = pl.pallas_call(
    _geom_body,
    grid=(NEB,),
    in_specs=[pl.BlockSpec((BE, H), lambda i: (i, 0))] * 2,
    out_specs=pl.BlockSpec((BE, 16), lambda i: (i, 0)),
    out_shape=jax.ShapeDtypeStruct((E, 16), jnp.float32),
)


def _edge_plain_body(ga_ref, gb_ref, g_ref, wd_ref, u_ref):
    pre = ga_ref[...] + gb_ref[...] + g_ref[:, 0:1] * wd_ref[...]
    u_ref[...] = jnp.maximum(pre, 0.0)


_edge_plain = pl.pallas_call(
    _edge_plain_body,
    grid=(NEB,),
    in_specs=[pl.BlockSpec((BE, H), lambda i: (i, 0)),
              pl.BlockSpec((BE, H), lambda i: (i, 0)),
              pl.BlockSpec((BE, 16), lambda i: (i, 0)),
              _full((1, H))],
    out_specs=pl.BlockSpec((BE, H), lambda i: (i, 0)),
    out_shape=jax.ShapeDtypeStruct((E, H), jnp.float32),
)


def _edge_t_body(ga_ref, gb_ref, g_ref, wd_ref, wec_ref, bec_ref, wc2_ref,
                 bc2_ref, u_ref, r_ref):
    pre = ga_ref[...] + gb_ref[...] + g_ref[:, 0:1] * wd_ref[...]
    u = jnp.maximum(pre, 0.0)
    u_ref[...] = u
    z = jnp.maximum(jnp.dot(u, wec_ref[...],
                            preferred_element_type=jnp.float32) + bec_ref[...],
                    0.0)
    t = jnp.sum(z * wc2_ref[...], axis=1, keepdims=True) + bc2_ref[...]
    r_ref[...] = jnp.concatenate(
        [g_ref[:, 1:2] * t, g_ref[:, 2:3] * t,
         jnp.zeros((g_ref.shape[0], H - 2), jnp.float32)], axis=1)


_edge_t = pl.pallas_call(
    _edge_t_body,
    grid=(NEB,),
    in_specs=[pl.BlockSpec((BE, H), lambda i: (i, 0)),
              pl.BlockSpec((BE, H), lambda i: (i, 0)),
              pl.BlockSpec((BE, 16), lambda i: (i, 0)),
              _full((1, H)), _full((H, H)), _full((1, H)), _full((1, H)),
              _full((1, 1))],
    out_specs=[pl.BlockSpec((BE, H), lambda i: (i, 0)),
               pl.BlockSpec((BE, H), lambda i: (i, 0))],
    out_shape=[jax.ShapeDtypeStruct((E, H), jnp.float32),
               jax.ShapeDtypeStruct((E, H), jnp.float32)],
)


def _make_node(last):
    def body(h_ref, u2_ref, c2_ref, *rest):
        if last:
            s2_ref = rest[0]
            rest = rest[1:]
        (we2_ref, be2_ref, wn1a_ref, wn1b_ref,
         bn1_ref, wn2_ref, bn2_ref, wA_ref, bA_ref, wB_ref, *outs) = rest
        h = h_ref[...]
        U = u2_ref[0] + u2_ref[1]
        cnt = c2_ref[0, :, 0:1] + c2_ref[1, :, 0:1]
        m = jnp.dot(U, we2_ref[...],
                    preferred_element_type=jnp.float32) + cnt * be2_ref[...]
        q = jnp.maximum(
            jnp.dot(h, wn1a_ref[...], preferred_element_type=jnp.float32)
            + jnp.dot(m, wn1b_ref[...], preferred_element_type=jnp.float32)
            + bn1_ref[...], 0.0)
        hn = h + jnp.dot(q, wn2_ref[...],
                         preferred_element_type=jnp.float32) + bn2_ref[...]
        outs[0][...] = hn
        if not last:
            outs[1][...] = jnp.dot(hn, wA_ref[...],
                                   preferred_element_type=jnp.float32) + bA_ref[...]
            outs[2][...] = jnp.dot(hn, wB_ref[...],
                                   preferred_element_type=jnp.float32)
        else:
            qv = jnp.maximum(jnp.dot(hn, wA_ref[...],
                                     preferred_element_type=jnp.float32)
                             + bA_ref[...], 0.0)
            vv = jnp.dot(qv, wB_ref[...], preferred_element_type=jnp.float32)
            s = s2_ref[0, :, 0:2] + s2_ref[1, :, 0:2]
            v = vv + s / jnp.maximum(cnt, 1.0)
            nrm = jnp.sqrt(jnp.sum(v * v, axis=1, keepdims=True))
            outs[1][...] = v / jnp.maximum(nrm, 1e-12)

    big_spec = pl.BlockSpec((NC, BN, H), lambda i: (0, i, 0))
    if last:
        out_specs = [pl.BlockSpec((BN, H), lambda i: (i, 0)),
                     pl.BlockSpec((BN, 2), lambda i: (i, 0))]
        out_shape = [jax.ShapeDtypeStruct((N, H), jnp.float32),
                     jax.ShapeDtypeStruct((N, 2), jnp.float32)]
        wB_spec = _full((H, 2))
        extra = [big_spec]
    else:
        out_specs = [pl.BlockSpec((BN, H), lambda i: (i, 0))] * 3
        out_shape = [jax.ShapeDtypeStruct((N, H), jnp.float32)] * 3
        wB_spec = _full((H, H))
        extra = []

    return pl.pallas_call(
        body,
        grid=(NNB,),
        in_specs=[pl.BlockSpec((BN, H), lambda i: (i, 0)),
                  big_spec, big_spec] + extra +
                 [_full((H, H)), _full((1, H)), _full((H, H)), _full((H, H)),
                  _full((1, H)), _full((H, H)), _full((1, H)),
                  _full((H, H)), _full((1, H)), wB_spec],
        out_specs=out_specs,
        out_shape=out_shape,
    )


_node_mid = _make_node(False)
_node_last = _make_node(True)


def kernel(h, x, edge_index, params):
    p = params
    row = edge_index[0]
    col = edge_index[1]

    W1a = p['W_e1'][:H]
    W1b = p['W_e1'][H:2 * H]
    wd = p['W_e1'][2 * H].reshape(1, H)
    W_ec = p['W_e2'] @ p['W_c1']
    b_ec = (p['b_e2'] @ p['W_c1'] + p['b_c1']).reshape(1, H)
    wc2 = p['W_c2'].reshape(1, H)
    bc2 = p['b_c2'].reshape(1, 1)
    be1 = p['b_e1'].reshape(1, H)
    be2 = p['b_e2'].reshape(1, H)
    Wn1a = p['W_n1'][:H]
    Wn1b = p['W_n1'][H:]
    bn1 = p['b_n1'].reshape(1, H)
    bn2 = p['b_n2'].reshape(1, H)
    bv1 = p['b_v1'].reshape(1, H)

    xp = jnp.pad(x, ((0, 0), (0, H - 2)))
    row3 = row.reshape(NW, NCHUNK, CHUNK)
    zH = jnp.zeros((8, H), jnp.float32)
    oH = jnp.ones((CHUNK, H), jnp.float32)

    h1, A, B = _init_call(h, p['W_emb'], p['b_emb'].reshape(1, H),
                          W1a, be1, W1b)
    xr, xc = _gather2_h(xp, xp, row, col)
    geom = _geom_call(xr, xc)

    zH_dep, _ = lax.optimization_barrier((zH, xr))
    (C2,) = _segsum_ones(row3, zH_dep, oH)

    A, _ = lax.optimization_barrier((A, C2))

    v = None
    for l in range(4):
        ga, gb = _gather2_h(A, B, row, col)
        if l < 3:
            u = _edge_plain(ga, gb, geom, wd)
        else:
            u, r128 = _edge_t(ga, gb, geom, wd, W_ec, b_ec, wc2, bc2)
        (U2,) = _segsum_plain(u, row3, zH)
        if l < 3:
            h1, A, B = _node_mid(h1, U2, C2, p['W_e2'], be2, Wn1a, Wn1b,
                                 bn1, p['W_n2'], bn2, W1a, be1, W1b)
        else:
            zH3, _ = lax.optimization_barrier((zH, U2))
            (S2,) = _segsum_plain(r128, row3, zH3)
            h1, v = _node_last(h1, U2, C2, S2, p['W_e2'], be2, Wn1a, Wn1b,
                               bn1, p['W_n2'], bn2, p['W_v1'], bv1, p['W_v2'])

    return (h1, x, v)
